# Initial kernel scaffold; baseline (speedup 1.0000x reference)
import functools

import jax
import jax.numpy as jnp
from jax import lax
from jax.experimental import pallas as pl
from jax.experimental.pallas import tpu as pltpu
from jax.experimental.pallas import tpu_sc as plsc

F32 = jnp.float32
I32 = jnp.int32

V = 10000
E = 160000
T = 4
SNUM = 3
D = 64
PROP = 16
SHARP = 5.0

NC = 2
NS = 16
NW = NC * NS

VP = 10240
VSLICE = VP // NS

CH = 128
EP = 163840
CPW = EP // (NW * CH)
NCHUNK = EP // CH

EB = 512
VB1 = 1024
VB3 = 256

HROW = T * D


def _relu(x):
    return jnp.maximum(x, 0.0)


def _enc_body(fh, fc, fp, wf1, bf1, wf2, bf2, wp1, bp1, wp2, bp2,
              wv1, bv1, wv2, bv2, wc1, bc1, wc2, bc2,
              h_o, th_o, v_o, hc_o):
    for t in range(T):
        x = _relu(jnp.dot(fh[t], wf1) + bf1)
        h_o[:, t, :] = jnp.dot(x, wf2) + bf2
    th = jnp.dot(_relu(jnp.dot(fp[...], wp1) + bp1), wp2) + bp2
    th_o[...] = th
    xv = jnp.dot(_relu(jnp.dot(th, wv1) + bv1), wv2) + bv2
    v_o[...] = jnp.logaddexp(xv, 0.0)
    hc_o[...] = jnp.dot(_relu(jnp.dot(fc[...], wc1) + bc1), wc2) + bc2


def _run_encoders(fh, fc, fp, wts):
    nblk = VP // VB1
    full = lambda a: pl.BlockSpec(a.shape, lambda i: (0,) * a.ndim)
    in_specs = [
        pl.BlockSpec((T, VB1, 8), lambda i: (0, i, 0)),
        pl.BlockSpec((VB1, 8), lambda i: (i, 0)),
        pl.BlockSpec((VB1, 8), lambda i: (i, 0)),
    ] + [full(w) for w in wts]
    out_specs = [
        pl.BlockSpec((VB1, T, D), lambda i: (i, 0, 0)),
        pl.BlockSpec((VB1, PROP), lambda i: (i, 0)),
        pl.BlockSpec((VB1, 1), lambda i: (i, 0)),
        pl.BlockSpec((VB1, D), lambda i: (i, 0)),
    ]
    out_shape = [
        jax.ShapeDtypeStruct((VP, T, D), F32),
        jax.ShapeDtypeStruct((VP, PROP), F32),
        jax.ShapeDtypeStruct((VP, 1), F32),
        jax.ShapeDtypeStruct((VP, D), F32),
    ]
    return pl.pallas_call(
        _enc_body, grid=(nblk,), in_specs=in_specs, out_specs=out_specs,
        out_shape=out_shape)(fh, fc, fp, *wts)


def _sc_gather(h_table, src_idx, dst_idx):
    mesh = plsc.VectorSubcoreMesh(core_axis_name="c", subcore_axis_name="s")

    @functools.partial(
        pl.kernel, mesh=mesh,
        out_type=(jax.ShapeDtypeStruct((SNUM, EP, HROW), F32),
                  jax.ShapeDtypeStruct((SNUM, EP, HROW), F32)),
        scratch_types=[
            pltpu.VMEM((CH,), I32),
            pltpu.VMEM((CH, HROW), F32),
            pltpu.SemaphoreType.DMA,
        ],
    )
    def gather_k(h_hbm, src_hbm, dst_hbm, gs_hbm, gd_hbm, idx_v, rows_v, sem):
        wid = lax.axis_index("s") * NC + lax.axis_index("c")
        base = wid * (CPW * CH)

        def job(idx_hbm, out_hbm, s):
            def chunk(c, carry):
                off = base + c * CH
                pltpu.sync_copy(idx_hbm.at[s, pl.ds(off, CH)], idx_v)
                pltpu.async_copy(h_hbm.at[idx_v], rows_v, sem).wait()
                pltpu.sync_copy(rows_v, out_hbm.at[s, pl.ds(off, CH)])
                return carry
            lax.fori_loop(0, CPW, chunk, 0)

        for s in range(SNUM):
            job(src_hbm, gs_hbm, s)
            job(dst_hbm, gd_hbm, s)

    return gather_k(h_table, src_idx, dst_idx)


def _edge_body(gs, gd, w1a, w1b, b1, w2, b2, w3, b3, out):
    e = pl.program_id(1)
    rows = e * EB + lax.broadcasted_iota(I32, (EB, 1), 0)
    valid = rows < E
    gsb = gs[0]
    gdb = gd[0]
    for t in range(T):
        hs = gsb[:, t * D:(t + 1) * D]
        hd = gdb[:, t * D:(t + 1) * D]
        x = _relu(jnp.dot(hs, w1a[0]) + jnp.dot(hd, w1b[0]) + b1[0])
        x = _relu(jnp.dot(x, w2[0]) + b2[0])
        m = jnp.dot(x, w3[0]) + b3[0]
        out[0, t] = jnp.where(valid, m, 0.0)


def _run_edge_mlp(gs, gd, w1a, w1b, b1, w2, b2, w3, b3):
    def wspec(a):
        return pl.BlockSpec((1,) + a.shape[1:],
                            lambda s, e: (s,) + (0,) * (a.ndim - 1))
    in_specs = [
        pl.BlockSpec((1, EB, HROW), lambda s, e: (s, e, 0)),
        pl.BlockSpec((1, EB, HROW), lambda s, e: (s, e, 0)),
        wspec(w1a), wspec(w1b), wspec(b1), wspec(w2), wspec(b2),
        wspec(w3), wspec(b3),
    ]
    out_specs = pl.BlockSpec((1, T, EB, D), lambda s, e: (s, 0, e, 0))
    return pl.pallas_call(
        _edge_body, grid=(SNUM, EP // EB), in_specs=in_specs,
        out_specs=out_specs,
        out_shape=jax.ShapeDtypeStruct((SNUM, T, EP, D), F32),
    )(gs, gd, w1a, w1b, b1, w2, b2, w3, b3)


def _sc_scatter(msgs, dst_chunks, zeros_slice):
    mesh = plsc.VectorSubcoreMesh(core_axis_name="c", subcore_axis_name="s")

    @functools.partial(
        pl.kernel, mesh=mesh,
        out_type=jax.ShapeDtypeStruct((NC, SNUM, T, VP, D), F32),
        scratch_types=[
            pltpu.VMEM((CH,), I32),
            pltpu.VMEM((CH, D), F32),
            pltpu.VMEM_SHARED((VP, D), F32),
        ],
    )
    def scatter_k(m_hbm, dst_hbm, zero_hbm, out_hbm, idx_v, msg_v, acc_sh):
        cid = lax.axis_index("c")
        sid = lax.axis_index("s")
        wid = sid * NC + cid
        myrow = sid * VSLICE
        for s in range(SNUM):
            for t in range(T):
                pltpu.sync_copy(zero_hbm, acc_sh.at[pl.ds(myrow, VSLICE)])
                plsc.subcore_barrier()

                def chunk(c, carry):
                    ci = wid * CPW + c
                    pltpu.sync_copy(dst_hbm.at[s, ci], idx_v)
                    pltpu.sync_copy(m_hbm.at[s, t, pl.ds(ci * CH, CH)], msg_v)
                    pltpu.sync_copy(msg_v, acc_sh.at[idx_v], add=True)
                    return carry
                lax.fori_loop(0, CPW, chunk, 0)
                plsc.subcore_barrier()
                pltpu.sync_copy(acc_sh.at[pl.ds(myrow, VSLICE)],
                                out_hbm.at[cid, s, t, pl.ds(myrow, VSLICE)])
                plsc.subcore_barrier()

    return scatter_k(msgs, dst_chunks, zeros_slice)


def _fin_body(hb, hcb, thb, vb, taub, pb,
              wn1, bn1, wn2, bn2,
              wah, wam, watau, wath, ba, wa2, ba2,
              wd1, bd1, wd2, bd2, wd3, bd3, wd4, bd4, wd5, bd5,
              out):
    agg = pb[0] + pb[1]

    msgs = []
    for t in range(T):
        row = []
        ht = hb[:, t, :]
        for s in range(SNUM):
            x = jnp.concatenate([ht, agg[s, t]], axis=-1)
            x = _relu(jnp.dot(x, wn1[s]) + bn1[s])
            row.append(jnp.dot(x, wn2[s]) + bn2[s])
        msgs.append(row)

    vv = vb[...]
    radius = vv * taub
    smax = jnp.zeros_like(radius)
    for k in range(1, SNUM + 1):
        smax = smax + jax.nn.sigmoid(SHARP * (radius - float(k)))
    smax = jnp.maximum(smax - 1.0, 0.0)
    logmask_cols = []
    for t in range(T):
        st = smax[:, t:t + 1]
        for s in range(SNUM):
            mk = jax.nn.sigmoid(SHARP * (st - float(s)))
            logmask_cols.append(jnp.log(mk + 1e-9))
    logmask = jnp.concatenate(logmask_cols, axis=-1)

    hcur = hcb[...]
    base = jnp.dot(hcur, wah) + jnp.dot(thb[...], wath) + ba
    logit_cols = []
    for t in range(T):
        bt = base + taub[:, t:t + 1] * watau
        for s in range(SNUM):
            z = _relu(bt + jnp.dot(msgs[t][s], wam))
            logit_cols.append(jnp.dot(z, wa2) + ba2)
    logits = jnp.concatenate(logit_cols, axis=-1)

    masked = logits + logmask
    mx = jnp.max(masked, axis=-1, keepdims=True)
    ex = jnp.exp(masked - mx)
    w = ex / jnp.sum(ex, axis=-1, keepdims=True)

    aggf = jnp.zeros((hcur.shape[0], D), F32)
    i = 0
    for t in range(T):
        for s in range(SNUM):
            aggf = aggf + w[:, i:i + 1] * msgs[t][s]
            i += 1

    x = jnp.concatenate([hcur, aggf], axis=-1)
    x = jnp.tanh(jnp.dot(x, wd1) + bd1)
    x = jnp.tanh(jnp.dot(x, wd2) + bd2)
    x = jnp.tanh(jnp.dot(x, wd3) + bd3)
    x = jnp.tanh(jnp.dot(x, wd4) + bd4)
    out[...] = jnp.dot(x, wd5) + bd5


def _run_finale(hh, hc, th, v, tau, partials, wts):
    nblk = VP // VB3
    full = lambda a: pl.BlockSpec(a.shape, lambda i: (0,) * a.ndim)
    in_specs = [
        pl.BlockSpec((VB3, T, D), lambda i: (i, 0, 0)),
        pl.BlockSpec((VB3, D), lambda i: (i, 0)),
        pl.BlockSpec((VB3, PROP), lambda i: (i, 0)),
        pl.BlockSpec((VB3, 1), lambda i: (i, 0)),
        full(tau),
        pl.BlockSpec((NC, SNUM, T, VB3, D), lambda i: (0, 0, 0, i, 0)),
    ] + [full(w) for w in wts]
    return pl.pallas_call(
        _fin_body, grid=(nblk,), in_specs=in_specs,
        out_specs=pl.BlockSpec((VB3, 3), lambda i: (i, 0)),
        out_shape=jax.ShapeDtypeStruct((VP, 3), F32),
    )(hh, hc, th, v, tau, partials, *wts)


def _pad_rows(a, rows):
    return jnp.pad(a, ((0, rows - a.shape[0]), (0, 0)))


def kernel(dynamic_hist, reference_hist, dynamic_curr, reference_curr,
           constraint, stiffness, mass, tau_values, params,
           edge_index_s0, edge_index_s1, edge_index_s2):
    fh = jnp.concatenate(
        [dynamic_hist, reference_hist,
         jnp.zeros((T, V, 2), F32)], axis=-1)
    fh = jnp.pad(fh, ((0, 0), (0, VP - V), (0, 0)))
    fc = _pad_rows(jnp.concatenate(
        [dynamic_curr, reference_curr, jnp.zeros((V, 2), F32)], axis=-1), VP)
    fp = _pad_rows(jnp.concatenate(
        [constraint, stiffness, mass, jnp.zeros((V, 5), F32)], axis=-1), VP)

    p = params
    (wf1, bf1), (wf2, bf2) = p['feat_enc']
    (wp1, bp1), (wp2, bp2) = p['prop_enc']
    (wv1, bv1), (wv2, bv2) = p['vel']
    (wc1, bc1), (wc2, bc2) = p['cur_enc']
    wf1 = jnp.pad(wf1, ((0, 2), (0, 0)))
    wp1 = jnp.pad(wp1, ((0, 5), (0, 0)))
    wc1 = jnp.pad(wc1, ((0, 2), (0, 0)))
    r2 = lambda b: b.reshape(1, -1)
    enc_wts = [wf1, r2(bf1), wf2, r2(bf2), wp1, r2(bp1), wp2, r2(bp2),
               wv1, r2(bv1), wv2, r2(bv2), wc1, r2(bc1), wc2, r2(bc2)]

    hh, th, v, hc = _run_encoders(fh, fc, fp, enc_wts)
    h_table = hh.reshape(VP, HROW)

    eidx = [edge_index_s0, edge_index_s1, edge_index_s2]
    src = jnp.stack([jnp.pad(e[0].astype(I32), (0, EP - E)) for e in eidx])
    dst = jnp.stack([jnp.pad(e[1].astype(I32), (0, EP - E)) for e in eidx])

    gs, gd = _sc_gather(h_table, src, dst)

    w1a = jnp.stack([p['msea'][s]['edge'][0][0][:D] for s in range(SNUM)])
    w1b = jnp.stack([p['msea'][s]['edge'][0][0][D:] for s in range(SNUM)])
    b1 = jnp.stack([r2(p['msea'][s]['edge'][0][1]) for s in range(SNUM)])
    w2 = jnp.stack([p['msea'][s]['edge'][1][0] for s in range(SNUM)])
    b2 = jnp.stack([r2(p['msea'][s]['edge'][1][1]) for s in range(SNUM)])
    w3 = jnp.stack([p['msea'][s]['edge'][2][0] for s in range(SNUM)])
    b3 = jnp.stack([r2(p['msea'][s]['edge'][2][1]) for s in range(SNUM)])

    msgs = _run_edge_mlp(gs, gd, w1a, w1b, b1, w2, b2, w3, b3)

    partials = _sc_scatter(msgs, dst.reshape(SNUM, NCHUNK, CH),
                           jnp.zeros((VSLICE, D), F32))

    wn1 = jnp.stack([p['msea'][s]['node'][0][0] for s in range(SNUM)])
    bn1 = jnp.stack([p['msea'][s]['node'][0][1] for s in range(SNUM)])[:, None, :]
    wn2 = jnp.stack([p['msea'][s]['node'][1][0] for s in range(SNUM)])
    bn2 = jnp.stack([p['msea'][s]['node'][1][1] for s in range(SNUM)])[:, None, :]
    wa, ba = p['attn'][0]
    wah = wa[:D]
    wam = wa[D:2 * D]
    watau = wa[2 * D:2 * D + 1]
    wath = wa[2 * D + 1:]
    wa2, ba2 = p['attn'][1]
    dyn = p['dyn']
    fin_wts = [wn1, bn1, wn2, bn2,
               wah, wam, watau, wath, r2(ba), wa2, r2(ba2),
               dyn[0][0], r2(dyn[0][1]), dyn[1][0], r2(dyn[1][1]),
               dyn[2][0], r2(dyn[2][1]), dyn[3][0], r2(dyn[3][1]),
               dyn[4][0], r2(dyn[4][1])]

    tau = tau_values.reshape(1, T).astype(F32)
    out = _run_finale(hh, hc, th, v, tau, partials, fin_wts)
    return out[:V]


# speedup vs baseline: 1.4903x; 1.4903x over previous
import functools

import jax
import jax.numpy as jnp
from jax import lax
from jax.experimental import pallas as pl
from jax.experimental.pallas import tpu as pltpu
from jax.experimental.pallas import tpu_sc as plsc

F32 = jnp.float32
I32 = jnp.int32

V = 10000
E = 160000
T = 4
SNUM = 3
D = 64
PROP = 16
SHARP = 5.0

NC = 2
NS = 16
NW = NC * NS

VP = 10240
VSLICE = VP // NS

CH = 128
EP = 163840
CPW = EP // (NW * CH)
NCHUNK = EP // CH

EB = 512
VB1 = 1024
VB3 = 256

HROW = T * D


def _dot(a, b):
    return jnp.dot(a, b, precision=lax.Precision.HIGHEST)


def _relu(x):
    return jnp.maximum(x, 0.0)


def _enc_body(fh, fc, fp, *refs):
    (wf1, bf1, wf2, bf2, wp1, bp1, wp2, bp2,
     wv1, bv1, wv2, bv2, wc1, bc1, wc2, bc2) = [r[...] for r in refs[:16]]
    h_o, th_o, v_o, hc_o = refs[16:]
    for t in range(T):
        x = _relu(_dot(fh[t], wf1) + bf1)
        h_o[:, t, :] = _dot(x, wf2) + bf2
    th = _dot(_relu(_dot(fp[...], wp1) + bp1), wp2) + bp2
    th_o[...] = th
    xv = _dot(_relu(_dot(th, wv1) + bv1), wv2) + bv2
    v_o[...] = jnp.logaddexp(xv, 0.0)
    hc_o[...] = _dot(_relu(_dot(fc[...], wc1) + bc1), wc2) + bc2


def _run_encoders(fh, fc, fp, wts):
    nblk = VP // VB1
    full = lambda a: pl.BlockSpec(a.shape, lambda i: (0,) * a.ndim)
    in_specs = [
        pl.BlockSpec((T, VB1, 8), lambda i: (0, i, 0)),
        pl.BlockSpec((VB1, 8), lambda i: (i, 0)),
        pl.BlockSpec((VB1, 8), lambda i: (i, 0)),
    ] + [full(w) for w in wts]
    out_specs = [
        pl.BlockSpec((VB1, T, D), lambda i: (i, 0, 0)),
        pl.BlockSpec((VB1, PROP), lambda i: (i, 0)),
        pl.BlockSpec((VB1, 1), lambda i: (i, 0)),
        pl.BlockSpec((VB1, D), lambda i: (i, 0)),
    ]
    out_shape = [
        jax.ShapeDtypeStruct((VP, T, D), F32),
        jax.ShapeDtypeStruct((VP, PROP), F32),
        jax.ShapeDtypeStruct((VP, 1), F32),
        jax.ShapeDtypeStruct((VP, D), F32),
    ]
    return pl.pallas_call(
        _enc_body, grid=(nblk,), in_specs=in_specs, out_specs=out_specs,
        out_shape=out_shape)(fh, fc, fp, *wts)


def _sc_gather(h_table, src_idx, dst_idx):
    mesh = plsc.VectorSubcoreMesh(core_axis_name="c", subcore_axis_name="s")

    @functools.partial(
        pl.kernel, mesh=mesh,
        out_type=(jax.ShapeDtypeStruct((SNUM, EP, HROW), F32),
                  jax.ShapeDtypeStruct((SNUM, EP, HROW), F32)),
        scratch_types=[
            pltpu.VMEM((CH,), I32),
            pltpu.VMEM((CH, HROW), F32),
            pltpu.SemaphoreType.DMA,
        ],
    )
    def gather_k(h_hbm, src_hbm, dst_hbm, gs_hbm, gd_hbm, idx_v, rows_v, sem):
        wid = lax.axis_index("s") * NC + lax.axis_index("c")
        base = wid * (CPW * CH)

        def job(idx_hbm, out_hbm, s):
            def chunk(c, carry):
                off = base + c * CH
                pltpu.sync_copy(idx_hbm.at[pl.ds(s * EP + off, CH)], idx_v)
                pltpu.async_copy(h_hbm.at[idx_v], rows_v, sem).wait()
                pltpu.sync_copy(rows_v, out_hbm.at[s, pl.ds(off, CH)])
                return carry
            lax.fori_loop(0, CPW, chunk, 0)

        for s in range(SNUM):
            job(src_hbm, gs_hbm, s)
            job(dst_hbm, gd_hbm, s)

    return gather_k(h_table, src_idx, dst_idx)


def _edge_body(gs, gd, w1a, w1b, b1, w2, b2, w3, b3, out):
    w1a, w1b, b1, w2, b2, w3, b3 = (
        w1a[...], w1b[...], b1[...], w2[...], b2[...], w3[...], b3[...])
    e = pl.program_id(1)
    rows = e * EB + lax.broadcasted_iota(I32, (EB, 1), 0)
    valid = rows < E
    gsb = gs[0]
    gdb = gd[0]
    ms = []
    for t in range(T):
        hs = gsb[:, t * D:(t + 1) * D]
        hd = gdb[:, t * D:(t + 1) * D]
        x = _relu(_dot(hs, w1a[0]) + _dot(hd, w1b[0]) + b1[0])
        x = _relu(_dot(x, w2[0]) + b2[0])
        m = _dot(x, w3[0]) + b3[0]
        ms.append(jnp.where(valid, m, 0.0))
    out[0, 0] = jnp.concatenate([ms[0], ms[1]], axis=-1)
    out[0, 1] = jnp.concatenate([ms[2], ms[3]], axis=-1)


def _run_edge_mlp(gs, gd, w1a, w1b, b1, w2, b2, w3, b3):
    def wspec(a):
        return pl.BlockSpec((1,) + a.shape[1:],
                            lambda s, e: (s,) + (0,) * (a.ndim - 1))
    in_specs = [
        pl.BlockSpec((1, EB, HROW), lambda s, e: (s, e, 0)),
        pl.BlockSpec((1, EB, HROW), lambda s, e: (s, e, 0)),
        wspec(w1a), wspec(w1b), wspec(b1), wspec(w2), wspec(b2),
        wspec(w3), wspec(b3),
    ]
    out_specs = pl.BlockSpec((1, 2, EB, 2 * D), lambda s, e: (s, 0, e, 0))
    return pl.pallas_call(
        _edge_body, grid=(SNUM, EP // EB), in_specs=in_specs,
        out_specs=out_specs,
        out_shape=jax.ShapeDtypeStruct((SNUM, 2, EP, 2 * D), F32),
    )(gs, gd, w1a, w1b, b1, w2, b2, w3, b3)


def _sc_scatter(msgs2d, dst_flat, zeros_row):
    mesh = plsc.VectorSubcoreMesh(core_axis_name="c", subcore_axis_name="s")

    @functools.partial(
        pl.kernel, mesh=mesh,
        out_type=jax.ShapeDtypeStruct((NC * SNUM * 2 * VP, 2 * D), F32),
        scratch_types=[
            pltpu.VMEM((CH,), I32),
            pltpu.VMEM((CH, 2 * D), F32),
            pltpu.VMEM((CH, 2 * D), F32),
            pltpu.VMEM_SHARED((VP, 2 * D), F32),
        ],
    )
    def scatter_k(m_hbm, dst_hbm, zero_hbm, out_hbm, idx_v, msg_v, zed_v,
                  acc_sh):
        cid = lax.axis_index("c")
        sid = lax.axis_index("s")
        wid = sid * NC + cid
        myrow = sid * VSLICE
        pltpu.sync_copy(zero_hbm, zed_v)
        for s in range(SNUM):
            for u in range(2):
                def zrow(z, carry):
                    pltpu.sync_copy(zed_v,
                                    acc_sh.at[pl.ds(myrow + z * CH, CH)])
                    return carry
                lax.fori_loop(0, VSLICE // CH, zrow, 0)
                plsc.subcore_barrier()

                def chunk(c, carry):
                    ci = wid * CPW + c
                    pltpu.sync_copy(dst_hbm.at[pl.ds(s * EP + ci * CH, CH)],
                                    idx_v)
                    pltpu.sync_copy(
                        m_hbm.at[pl.ds((s * 2 + u) * EP + ci * CH, CH)],
                        msg_v)
                    pltpu.sync_copy(msg_v, acc_sh.at[idx_v], add=True)
                    return carry
                lax.fori_loop(0, CPW, chunk, 0)
                plsc.subcore_barrier()
                pltpu.sync_copy(
                    acc_sh.at[pl.ds(myrow, VSLICE)],
                    out_hbm.at[pl.ds(((cid * SNUM + s) * 2 + u) * VP + myrow,
                                     VSLICE)])
                plsc.subcore_barrier()

    return scatter_k(msgs2d, dst_flat, zeros_row)


def _fin_body(hb, hcb, thb, vb, taub, pb, *refs):
    (wn1, bn1, wn2, bn2,
     wah, wam, watau, wath, ba, wa2, ba2,
     wd1, bd1, wd2, bd2, wd3, bd3, wd4, bd4, wd5, bd5) = [
        r[...] for r in refs[:21]]
    out = refs[21]
    taub = taub[...]
    agg2 = pb[0] + pb[1]

    msgs = []
    for t in range(T):
        row = []
        ht = hb[:, t, :]
        for s in range(SNUM):
            a = agg2[s, t // 2][:, (t % 2) * D:(t % 2 + 1) * D]
            x = jnp.concatenate([ht, a], axis=-1)
            x = _relu(_dot(x, wn1[s]) + bn1[s])
            row.append(_dot(x, wn2[s]) + bn2[s])
        msgs.append(row)

    vv = vb[...]
    radius = vv * taub
    smax = jnp.zeros_like(radius)
    for k in range(1, SNUM + 1):
        smax = smax + jax.nn.sigmoid(SHARP * (radius - float(k)))
    smax = jnp.maximum(smax - 1.0, 0.0)
    logmask_cols = []
    for t in range(T):
        st = smax[:, t:t + 1]
        for s in range(SNUM):
            mk = jax.nn.sigmoid(SHARP * (st - float(s)))
            logmask_cols.append(jnp.log(mk + 1e-9))
    logmask = jnp.concatenate(logmask_cols, axis=-1)

    hcur = hcb[...]
    base = _dot(hcur, wah) + _dot(thb[...], wath) + ba
    logit_cols = []
    for t in range(T):
        bt = base + taub[:, t:t + 1] * watau
        for s in range(SNUM):
            z = _relu(bt + _dot(msgs[t][s], wam))
            logit_cols.append(_dot(z, wa2) + ba2)
    logits = jnp.concatenate(logit_cols, axis=-1)

    masked = logits + logmask
    mx = jnp.max(masked, axis=-1, keepdims=True)
    ex = jnp.exp(masked - mx)
    w = ex / jnp.sum(ex, axis=-1, keepdims=True)

    aggf = jnp.zeros((hcur.shape[0], D), F32)
    i = 0
    for t in range(T):
        for s in range(SNUM):
            aggf = aggf + w[:, i:i + 1] * msgs[t][s]
            i += 1

    x = jnp.concatenate([hcur, aggf], axis=-1)
    x = jnp.tanh(_dot(x, wd1) + bd1)
    x = jnp.tanh(_dot(x, wd2) + bd2)
    x = jnp.tanh(_dot(x, wd3) + bd3)
    x = jnp.tanh(_dot(x, wd4) + bd4)
    out[...] = _dot(x, wd5) + bd5


def _run_finale(hh, hc, th, v, tau, partials, wts):
    nblk = VP // VB3
    full = lambda a: pl.BlockSpec(a.shape, lambda i: (0,) * a.ndim)
    in_specs = [
        pl.BlockSpec((VB3, T, D), lambda i: (i, 0, 0)),
        pl.BlockSpec((VB3, D), lambda i: (i, 0)),
        pl.BlockSpec((VB3, PROP), lambda i: (i, 0)),
        pl.BlockSpec((VB3, 1), lambda i: (i, 0)),
        full(tau),
        pl.BlockSpec((NC, SNUM, 2, VB3, 2 * D), lambda i: (0, 0, 0, i, 0)),
    ] + [full(w) for w in wts]
    return pl.pallas_call(
        _fin_body, grid=(nblk,), in_specs=in_specs,
        out_specs=pl.BlockSpec((VB3, 3), lambda i: (i, 0)),
        out_shape=jax.ShapeDtypeStruct((VP, 3), F32),
    )(hh, hc, th, v, tau, partials, *wts)


def _pad_rows(a, rows):
    return jnp.pad(a, ((0, rows - a.shape[0]), (0, 0)))


def kernel(dynamic_hist, reference_hist, dynamic_curr, reference_curr,
           constraint, stiffness, mass, tau_values, params,
           edge_index_s0, edge_index_s1, edge_index_s2):
    fh = jnp.concatenate(
        [dynamic_hist, reference_hist,
         jnp.zeros((T, V, 2), F32)], axis=-1)
    fh = jnp.pad(fh, ((0, 0), (0, VP - V), (0, 0)))
    fc = _pad_rows(jnp.concatenate(
        [dynamic_curr, reference_curr, jnp.zeros((V, 2), F32)], axis=-1), VP)
    fp = _pad_rows(jnp.concatenate(
        [constraint, stiffness, mass, jnp.zeros((V, 5), F32)], axis=-1), VP)

    p = params
    (wf1, bf1), (wf2, bf2) = p['feat_enc']
    (wp1, bp1), (wp2, bp2) = p['prop_enc']
    (wv1, bv1), (wv2, bv2) = p['vel']
    (wc1, bc1), (wc2, bc2) = p['cur_enc']
    wf1 = jnp.pad(wf1, ((0, 2), (0, 0)))
    wp1 = jnp.pad(wp1, ((0, 5), (0, 0)))
    wc1 = jnp.pad(wc1, ((0, 2), (0, 0)))
    r2 = lambda b: b.reshape(1, -1)
    enc_wts = [wf1, r2(bf1), wf2, r2(bf2), wp1, r2(bp1), wp2, r2(bp2),
               wv1, r2(bv1), wv2, r2(bv2), wc1, r2(bc1), wc2, r2(bc2)]

    hh, th, v, hc = _run_encoders(fh, fc, fp, enc_wts)
    h_table = hh.reshape(VP, HROW)

    eidx = [edge_index_s0, edge_index_s1, edge_index_s2]
    src = jnp.stack([jnp.pad(e[0].astype(I32), (0, EP - E)) for e in eidx])
    dst = jnp.stack([jnp.pad(e[1].astype(I32), (0, EP - E)) for e in eidx])

    gs, gd = _sc_gather(h_table, src.reshape(-1), dst.reshape(-1))

    w1a = jnp.stack([p['msea'][s]['edge'][0][0][:D] for s in range(SNUM)])
    w1b = jnp.stack([p['msea'][s]['edge'][0][0][D:] for s in range(SNUM)])
    b1 = jnp.stack([r2(p['msea'][s]['edge'][0][1]) for s in range(SNUM)])
    w2 = jnp.stack([p['msea'][s]['edge'][1][0] for s in range(SNUM)])
    b2 = jnp.stack([r2(p['msea'][s]['edge'][1][1]) for s in range(SNUM)])
    w3 = jnp.stack([p['msea'][s]['edge'][2][0] for s in range(SNUM)])
    b3 = jnp.stack([r2(p['msea'][s]['edge'][2][1]) for s in range(SNUM)])

    msgs = _run_edge_mlp(gs, gd, w1a, w1b, b1, w2, b2, w3, b3)

    partials = _sc_scatter(msgs.reshape(SNUM * 2 * EP, 2 * D),
                           dst.reshape(-1), jnp.zeros((CH, 2 * D), F32))
    partials = partials.reshape(NC, SNUM, 2, VP, 2 * D)

    wn1 = jnp.stack([p['msea'][s]['node'][0][0] for s in range(SNUM)])
    bn1 = jnp.stack([p['msea'][s]['node'][0][1] for s in range(SNUM)])[:, None, :]
    wn2 = jnp.stack([p['msea'][s]['node'][1][0] for s in range(SNUM)])
    bn2 = jnp.stack([p['msea'][s]['node'][1][1] for s in range(SNUM)])[:, None, :]
    wa, ba = p['attn'][0]
    wah = wa[:D]
    wam = wa[D:2 * D]
    watau = wa[2 * D:2 * D + 1]
    wath = wa[2 * D + 1:]
    wa2, ba2 = p['attn'][1]
    dyn = p['dyn']
    fin_wts = [wn1, bn1, wn2, bn2,
               wah, wam, watau, wath, r2(ba), wa2, r2(ba2),
               dyn[0][0], r2(dyn[0][1]), dyn[1][0], r2(dyn[1][1]),
               dyn[2][0], r2(dyn[2][1]), dyn[3][0], r2(dyn[3][1]),
               dyn[4][0], r2(dyn[4][1])]

    tau = tau_values.reshape(1, T).astype(F32)
    out = _run_finale(hh, hc, th, v, tau, partials, fin_wts)
    return out[:V]


# speedup vs baseline: 2.5580x; 1.7164x over previous
import functools

import jax
import jax.numpy as jnp
from jax import lax
from jax.experimental import pallas as pl
from jax.experimental.pallas import tpu as pltpu
from jax.experimental.pallas import tpu_sc as plsc

F32 = jnp.float32
I32 = jnp.int32

V = 10000
E = 160000
T = 4
SNUM = 3
D = 64
PROP = 16
SHARP = 5.0

NC = 2
NS = 16
NW = NC * NS

VP = 10240
VSLICE = VP // NS

CH = 128
EP = 163840
CPW = EP // (NW * CH)
NCHUNK = EP // CH

EB = 512
VB1 = 1024
VB3 = 256

HROW = T * D


def _dot(a, b):
    return jnp.dot(a, b)


def _relu(x):
    return jnp.maximum(x, 0.0)


def _enc_body(fh, fc, fp, *refs):
    (wf1, bf1, wf2, bf2, wp1, bp1, wp2, bp2,
     wv1, bv1, wv2, bv2, wc1, bc1, wc2, bc2) = [r[...] for r in refs[:16]]
    h_o, th_o, v_o, hc_o = refs[16:]
    for t in range(T):
        x = _relu(_dot(fh[t], wf1) + bf1)
        h_o[:, t, :] = _dot(x, wf2) + bf2
    th = _dot(_relu(_dot(fp[...], wp1) + bp1), wp2) + bp2
    th_o[...] = th
    xv = _dot(_relu(_dot(th, wv1) + bv1), wv2) + bv2
    v_o[...] = jnp.logaddexp(xv, 0.0)
    hc_o[...] = _dot(_relu(_dot(fc[...], wc1) + bc1), wc2) + bc2


def _run_encoders(fh, fc, fp, wts):
    nblk = VP // VB1
    full = lambda a: pl.BlockSpec(a.shape, lambda i: (0,) * a.ndim)
    in_specs = [
        pl.BlockSpec((T, VB1, 8), lambda i: (0, i, 0)),
        pl.BlockSpec((VB1, 8), lambda i: (i, 0)),
        pl.BlockSpec((VB1, 8), lambda i: (i, 0)),
    ] + [full(w) for w in wts]
    out_specs = [
        pl.BlockSpec((VB1, T, D), lambda i: (i, 0, 0)),
        pl.BlockSpec((VB1, PROP), lambda i: (i, 0)),
        pl.BlockSpec((VB1, 1), lambda i: (i, 0)),
        pl.BlockSpec((VB1, D), lambda i: (i, 0)),
    ]
    out_shape = [
        jax.ShapeDtypeStruct((VP, T, D), F32),
        jax.ShapeDtypeStruct((VP, PROP), F32),
        jax.ShapeDtypeStruct((VP, 1), F32),
        jax.ShapeDtypeStruct((VP, D), F32),
    ]
    return pl.pallas_call(
        _enc_body, grid=(nblk,), in_specs=in_specs, out_specs=out_specs,
        out_shape=out_shape)(fh, fc, fp, *wts)


def _sc_gather(h_table, src_idx, dst_idx):
    mesh = plsc.VectorSubcoreMesh(core_axis_name="c", subcore_axis_name="s")

    @functools.partial(
        pl.kernel, mesh=mesh,
        out_type=(jax.ShapeDtypeStruct((SNUM, EP, HROW), F32),
                  jax.ShapeDtypeStruct((SNUM, EP, HROW), F32)),
        scratch_types=[
            pltpu.VMEM((CH,), I32),
            pltpu.VMEM((CH, HROW), F32),
            pltpu.SemaphoreType.DMA,
        ],
    )
    def gather_k(h_hbm, src_hbm, dst_hbm, gs_hbm, gd_hbm, idx_v, rows_v, sem):
        wid = lax.axis_index("s") * NC + lax.axis_index("c")
        base = wid * (CPW * CH)

        def job(idx_hbm, out_hbm, s):
            def chunk(c, carry):
                off = base + c * CH
                pltpu.sync_copy(idx_hbm.at[pl.ds(s * EP + off, CH)], idx_v)
                pltpu.async_copy(h_hbm.at[idx_v], rows_v, sem).wait()
                pltpu.sync_copy(rows_v, out_hbm.at[s, pl.ds(off, CH)])
                return carry
            lax.fori_loop(0, CPW, chunk, 0)

        for s in range(SNUM):
            job(src_hbm, gs_hbm, s)
            job(dst_hbm, gd_hbm, s)

    return gather_k(h_table, src_idx, dst_idx)


def _edge_body(gs, gd, w1a, w1b, b1, w2, b2, w3, b3, out):
    w1a, w1b, b1, w2, b2, w3, b3 = (
        w1a[...], w1b[...], b1[...], w2[...], b2[...], w3[...], b3[...])
    e = pl.program_id(1)
    rows = e * EB + lax.broadcasted_iota(I32, (EB, 1), 0)
    valid = rows < E
    gsb = gs[0]
    gdb = gd[0]
    ms = []
    for t in range(T):
        hs = gsb[:, t * D:(t + 1) * D]
        hd = gdb[:, t * D:(t + 1) * D]
        x = _relu(_dot(hs, w1a[0]) + _dot(hd, w1b[0]) + b1[0])
        x = _relu(_dot(x, w2[0]) + b2[0])
        m = _dot(x, w3[0]) + b3[0]
        ms.append(jnp.where(valid, m, 0.0))
    out[0, 0] = jnp.concatenate([ms[0], ms[1]], axis=-1)
    out[0, 1] = jnp.concatenate([ms[2], ms[3]], axis=-1)


def _run_edge_mlp(gs, gd, w1a, w1b, b1, w2, b2, w3, b3):
    def wspec(a):
        return pl.BlockSpec((1,) + a.shape[1:],
                            lambda s, e: (s,) + (0,) * (a.ndim - 1))
    in_specs = [
        pl.BlockSpec((1, EB, HROW), lambda s, e: (s, e, 0)),
        pl.BlockSpec((1, EB, HROW), lambda s, e: (s, e, 0)),
        wspec(w1a), wspec(w1b), wspec(b1), wspec(w2), wspec(b2),
        wspec(w3), wspec(b3),
    ]
    out_specs = pl.BlockSpec((1, 2, EB, 2 * D), lambda s, e: (s, 0, e, 0))
    return pl.pallas_call(
        _edge_body, grid=(SNUM, EP // EB), in_specs=in_specs,
        out_specs=out_specs,
        out_shape=jax.ShapeDtypeStruct((SNUM, 2, EP, 2 * D), F32),
    )(gs, gd, w1a, w1b, b1, w2, b2, w3, b3)


def _sc_scatter(msgs2d, dst_flat, zeros_row):
    mesh = plsc.VectorSubcoreMesh(core_axis_name="c", subcore_axis_name="s")

    @functools.partial(
        pl.kernel, mesh=mesh,
        out_type=jax.ShapeDtypeStruct((NC * SNUM * 2 * VP, 2 * D), F32),
        scratch_types=[
            pltpu.VMEM((CH,), I32),
            pltpu.VMEM((CH, 2 * D), F32),
            pltpu.VMEM((CH, 2 * D), F32),
            pltpu.VMEM_SHARED((VP, 2 * D), F32),
        ],
    )
    def scatter_k(m_hbm, dst_hbm, zero_hbm, out_hbm, idx_v, msg_v, zed_v,
                  acc_sh):
        cid = lax.axis_index("c")
        sid = lax.axis_index("s")
        wid = sid * NC + cid
        myrow = sid * VSLICE
        pltpu.sync_copy(zero_hbm, zed_v)
        for s in range(SNUM):
            for u in range(2):
                def zrow(z, carry):
                    pltpu.sync_copy(zed_v,
                                    acc_sh.at[pl.ds(myrow + z * CH, CH)])
                    return carry
                lax.fori_loop(0, VSLICE // CH, zrow, 0)
                plsc.subcore_barrier()

                def chunk(c, carry):
                    ci = wid * CPW + c
                    pltpu.sync_copy(dst_hbm.at[pl.ds(s * EP + ci * CH, CH)],
                                    idx_v)
                    pltpu.sync_copy(
                        m_hbm.at[pl.ds((s * 2 + u) * EP + ci * CH, CH)],
                        msg_v)
                    pltpu.sync_copy(msg_v, acc_sh.at[idx_v], add=True)
                    return carry
                lax.fori_loop(0, CPW, chunk, 0)
                plsc.subcore_barrier()
                pltpu.sync_copy(
                    acc_sh.at[pl.ds(myrow, VSLICE)],
                    out_hbm.at[pl.ds(((cid * SNUM + s) * 2 + u) * VP + myrow,
                                     VSLICE)])
                plsc.subcore_barrier()

    return scatter_k(msgs2d, dst_flat, zeros_row)


def _fin_body(hb, hcb, thb, vb, taub, pb, *refs):
    (wn1, bn1, wn2, bn2,
     wah, wam, watau, wath, ba, wa2, ba2,
     wd1, bd1, wd2, bd2, wd3, bd3, wd4, bd4, wd5, bd5) = [
        r[...] for r in refs[:21]]
    out = refs[21]
    taub = taub[...]
    agg2 = pb[0] + pb[1]

    msgs = []
    for t in range(T):
        row = []
        ht = hb[:, t, :]
        for s in range(SNUM):
            a = agg2[s, t // 2][:, (t % 2) * D:(t % 2 + 1) * D]
            x = jnp.concatenate([ht, a], axis=-1)
            x = _relu(_dot(x, wn1[s]) + bn1[s])
            row.append(_dot(x, wn2[s]) + bn2[s])
        msgs.append(row)

    vv = vb[...]
    radius = vv * taub
    smax = jnp.zeros_like(radius)
    for k in range(1, SNUM + 1):
        smax = smax + jax.nn.sigmoid(SHARP * (radius - float(k)))
    smax = jnp.maximum(smax - 1.0, 0.0)
    logmask_cols = []
    for t in range(T):
        st = smax[:, t:t + 1]
        for s in range(SNUM):
            mk = jax.nn.sigmoid(SHARP * (st - float(s)))
            logmask_cols.append(jnp.log(mk + 1e-9))
    logmask = jnp.concatenate(logmask_cols, axis=-1)

    hcur = hcb[...]
    base = _dot(hcur, wah) + _dot(thb[...], wath) + ba
    logit_cols = []
    for t in range(T):
        bt = base + taub[:, t:t + 1] * watau
        for s in range(SNUM):
            z = _relu(bt + _dot(msgs[t][s], wam))
            logit_cols.append(_dot(z, wa2) + ba2)
    logits = jnp.concatenate(logit_cols, axis=-1)

    masked = logits + logmask
    mx = jnp.max(masked, axis=-1, keepdims=True)
    ex = jnp.exp(masked - mx)
    w = ex / jnp.sum(ex, axis=-1, keepdims=True)

    aggf = jnp.zeros((hcur.shape[0], D), F32)
    i = 0
    for t in range(T):
        for s in range(SNUM):
            aggf = aggf + w[:, i:i + 1] * msgs[t][s]
            i += 1

    x = jnp.concatenate([hcur, aggf], axis=-1)
    x = jnp.tanh(_dot(x, wd1) + bd1)
    x = jnp.tanh(_dot(x, wd2) + bd2)
    x = jnp.tanh(_dot(x, wd3) + bd3)
    x = jnp.tanh(_dot(x, wd4) + bd4)
    out[...] = _dot(x, wd5) + bd5


def _run_finale(hh, hc, th, v, tau, partials, wts):
    nblk = VP // VB3
    full = lambda a: pl.BlockSpec(a.shape, lambda i: (0,) * a.ndim)
    in_specs = [
        pl.BlockSpec((VB3, T, D), lambda i: (i, 0, 0)),
        pl.BlockSpec((VB3, D), lambda i: (i, 0)),
        pl.BlockSpec((VB3, PROP), lambda i: (i, 0)),
        pl.BlockSpec((VB3, 1), lambda i: (i, 0)),
        full(tau),
        pl.BlockSpec((NC, SNUM, 2, VB3, 2 * D), lambda i: (0, 0, 0, i, 0)),
    ] + [full(w) for w in wts]
    return pl.pallas_call(
        _fin_body, grid=(nblk,), in_specs=in_specs,
        out_specs=pl.BlockSpec((VB3, 3), lambda i: (i, 0)),
        out_shape=jax.ShapeDtypeStruct((VP, 3), F32),
    )(hh, hc, th, v, tau, partials, *wts)


def _pad_rows(a, rows):
    return jnp.pad(a, ((0, rows - a.shape[0]), (0, 0)))


def kernel(dynamic_hist, reference_hist, dynamic_curr, reference_curr,
           constraint, stiffness, mass, tau_values, params,
           edge_index_s0, edge_index_s1, edge_index_s2):
    fh = jnp.concatenate(
        [dynamic_hist, reference_hist,
         jnp.zeros((T, V, 2), F32)], axis=-1)
    fh = jnp.pad(fh, ((0, 0), (0, VP - V), (0, 0)))
    fc = _pad_rows(jnp.concatenate(
        [dynamic_curr, reference_curr, jnp.zeros((V, 2), F32)], axis=-1), VP)
    fp = _pad_rows(jnp.concatenate(
        [constraint, stiffness, mass, jnp.zeros((V, 5), F32)], axis=-1), VP)

    p = params
    (wf1, bf1), (wf2, bf2) = p['feat_enc']
    (wp1, bp1), (wp2, bp2) = p['prop_enc']
    (wv1, bv1), (wv2, bv2) = p['vel']
    (wc1, bc1), (wc2, bc2) = p['cur_enc']
    wf1 = jnp.pad(wf1, ((0, 2), (0, 0)))
    wp1 = jnp.pad(wp1, ((0, 5), (0, 0)))
    wc1 = jnp.pad(wc1, ((0, 2), (0, 0)))
    r2 = lambda b: b.reshape(1, -1)
    enc_wts = [wf1, r2(bf1), wf2, r2(bf2), wp1, r2(bp1), wp2, r2(bp2),
               wv1, r2(bv1), wv2, r2(bv2), wc1, r2(bc1), wc2, r2(bc2)]

    hh, th, v, hc = _run_encoders(fh, fc, fp, enc_wts)
    h_table = hh.reshape(VP, HROW)

    eidx = [edge_index_s0, edge_index_s1, edge_index_s2]
    src = jnp.stack([jnp.pad(e[0].astype(I32), (0, EP - E)) for e in eidx])
    dst = jnp.stack([jnp.pad(e[1].astype(I32), (0, EP - E)) for e in eidx])

    gs, gd = _sc_gather(h_table, src.reshape(-1), dst.reshape(-1))

    w1a = jnp.stack([p['msea'][s]['edge'][0][0][:D] for s in range(SNUM)])
    w1b = jnp.stack([p['msea'][s]['edge'][0][0][D:] for s in range(SNUM)])
    b1 = jnp.stack([r2(p['msea'][s]['edge'][0][1]) for s in range(SNUM)])
    w2 = jnp.stack([p['msea'][s]['edge'][1][0] for s in range(SNUM)])
    b2 = jnp.stack([r2(p['msea'][s]['edge'][1][1]) for s in range(SNUM)])
    w3 = jnp.stack([p['msea'][s]['edge'][2][0] for s in range(SNUM)])
    b3 = jnp.stack([r2(p['msea'][s]['edge'][2][1]) for s in range(SNUM)])

    msgs = _run_edge_mlp(gs, gd, w1a, w1b, b1, w2, b2, w3, b3)

    partials = _sc_scatter(msgs.reshape(SNUM * 2 * EP, 2 * D),
                           dst.reshape(-1), jnp.zeros((CH, 2 * D), F32))
    partials = partials.reshape(NC, SNUM, 2, VP, 2 * D)

    wn1 = jnp.stack([p['msea'][s]['node'][0][0] for s in range(SNUM)])
    bn1 = jnp.stack([p['msea'][s]['node'][0][1] for s in range(SNUM)])[:, None, :]
    wn2 = jnp.stack([p['msea'][s]['node'][1][0] for s in range(SNUM)])
    bn2 = jnp.stack([p['msea'][s]['node'][1][1] for s in range(SNUM)])[:, None, :]
    wa, ba = p['attn'][0]
    wah = wa[:D]
    wam = wa[D:2 * D]
    watau = wa[2 * D:2 * D + 1]
    wath = wa[2 * D + 1:]
    wa2, ba2 = p['attn'][1]
    dyn = p['dyn']
    fin_wts = [wn1, bn1, wn2, bn2,
               wah, wam, watau, wath, r2(ba), wa2, r2(ba2),
               dyn[0][0], r2(dyn[0][1]), dyn[1][0], r2(dyn[1][1]),
               dyn[2][0], r2(dyn[2][1]), dyn[3][0], r2(dyn[3][1]),
               dyn[4][0], r2(dyn[4][1])]

    tau = tau_values.reshape(1, T).astype(F32)
    out = _run_finale(hh, hc, th, v, tau, partials, fin_wts)
    return out[:V]


# speedup vs baseline: 2.6977x; 1.0546x over previous
import functools

import jax
import jax.numpy as jnp
from jax import lax
from jax.experimental import pallas as pl
from jax.experimental.pallas import tpu as pltpu
from jax.experimental.pallas import tpu_sc as plsc

F32 = jnp.float32
I32 = jnp.int32

V = 10000
E = 160000
T = 4
SNUM = 3
D = 64
PROP = 16
SHARP = 5.0

NC = 2
NS = 16
NW = NC * NS

VP = 10240
VSLICE = VP // NS

CH = 128
EP = 163840
CPW = EP // (NW * CH)
NCHUNK = EP // CH

EB = 512
VB1 = 1024
VB3 = 256

HROW = T * D


def _dot(a, b):
    return jnp.dot(a, b)


def _relu(x):
    return jnp.maximum(x, 0.0)


def _enc_body(fh, fc, fp, *refs):
    (wf1, bf1, wf2, bf2, wp1, bp1, wp2, bp2,
     wv1, bv1, wv2, bv2, wc1, bc1, wc2, bc2) = [r[...] for r in refs[:16]]
    h_o, th_o, v_o, hc_o = refs[16:]
    for t in range(T):
        x = _relu(_dot(fh[t], wf1) + bf1)
        h_o[:, t, :] = _dot(x, wf2) + bf2
    th = _dot(_relu(_dot(fp[...], wp1) + bp1), wp2) + bp2
    th_o[...] = th
    xv = _dot(_relu(_dot(th, wv1) + bv1), wv2) + bv2
    v_o[...] = jnp.logaddexp(xv, 0.0)
    hc_o[...] = _dot(_relu(_dot(fc[...], wc1) + bc1), wc2) + bc2


def _run_encoders(fh, fc, fp, wts):
    nblk = VP // VB1
    full = lambda a: pl.BlockSpec(a.shape, lambda i: (0,) * a.ndim)
    in_specs = [
        pl.BlockSpec((T, VB1, 8), lambda i: (0, i, 0)),
        pl.BlockSpec((VB1, 8), lambda i: (i, 0)),
        pl.BlockSpec((VB1, 8), lambda i: (i, 0)),
    ] + [full(w) for w in wts]
    out_specs = [
        pl.BlockSpec((VB1, T, D), lambda i: (i, 0, 0)),
        pl.BlockSpec((VB1, PROP), lambda i: (i, 0)),
        pl.BlockSpec((VB1, 1), lambda i: (i, 0)),
        pl.BlockSpec((VB1, D), lambda i: (i, 0)),
    ]
    out_shape = [
        jax.ShapeDtypeStruct((VP, T, D), F32),
        jax.ShapeDtypeStruct((VP, PROP), F32),
        jax.ShapeDtypeStruct((VP, 1), F32),
        jax.ShapeDtypeStruct((VP, D), F32),
    ]
    return pl.pallas_call(
        _enc_body, grid=(nblk,), in_specs=in_specs, out_specs=out_specs,
        out_shape=out_shape)(fh, fc, fp, *wts)


def _sc_gather(h_table, src_idx, dst_idx):
    mesh = plsc.VectorSubcoreMesh(core_axis_name="c", subcore_axis_name="s")

    @functools.partial(
        pl.kernel, mesh=mesh,
        out_type=(jax.ShapeDtypeStruct((SNUM, EP, HROW), F32),
                  jax.ShapeDtypeStruct((SNUM, EP, HROW), F32)),
        scratch_types=[
            pltpu.VMEM((CH,), I32),
            pltpu.VMEM((CH,), I32),
            pltpu.VMEM((CH, HROW), F32),
            pltpu.VMEM((CH, HROW), F32),
            pltpu.SemaphoreType.DMA,
            pltpu.SemaphoreType.DMA,
        ],
    )
    def gather_k(h_hbm, src_hbm, dst_hbm, gs_hbm, gd_hbm,
                 idx_a, idx_b, rows_a, rows_b, sem_a, sem_b):
        wid = lax.axis_index("s") * NC + lax.axis_index("c")
        base = wid * (CPW * CH)
        lastoff = base + (CPW - 2) * CH

        def job(idx_hbm, out_hbm, s):
            pltpu.sync_copy(idx_hbm.at[pl.ds(s * EP + base, CH)], idx_a)
            ga = pltpu.async_copy(h_hbm.at[idx_a], rows_a, sem_a)

            def pair(cc, carry):
                off0 = base + (2 * cc) * CH
                off1 = off0 + CH
                off2 = jnp.minimum(off0 + 2 * CH, lastoff)
                pltpu.sync_copy(idx_hbm.at[pl.ds(s * EP + off1, CH)], idx_b)
                pltpu.make_async_copy(h_hbm.at[idx_a], rows_a, sem_a).wait()
                pltpu.async_copy(h_hbm.at[idx_b], rows_b, sem_b)
                pltpu.sync_copy(rows_a, out_hbm.at[s, pl.ds(off0, CH)])
                pltpu.sync_copy(idx_hbm.at[pl.ds(s * EP + off2, CH)], idx_a)
                pltpu.make_async_copy(h_hbm.at[idx_b], rows_b, sem_b).wait()
                pltpu.async_copy(h_hbm.at[idx_a], rows_a, sem_a)
                pltpu.sync_copy(rows_b, out_hbm.at[s, pl.ds(off1, CH)])
                return carry
            lax.fori_loop(0, CPW // 2, pair, 0)
            pltpu.make_async_copy(h_hbm.at[idx_a], rows_a, sem_a).wait()

        for s in range(SNUM):
            job(src_hbm, gs_hbm, s)
            job(dst_hbm, gd_hbm, s)

    return gather_k(h_table, src_idx, dst_idx)


def _edge_body(gs, gd, w1a, w1b, b1, w2, b2, w3, b3, out):
    w1a, w1b, b1, w2, b2, w3, b3 = (
        w1a[...], w1b[...], b1[...], w2[...], b2[...], w3[...], b3[...])
    e = pl.program_id(1)
    rows = e * EB + lax.broadcasted_iota(I32, (EB, 1), 0)
    valid = rows < E
    gsb = gs[0]
    gdb = gd[0]
    ms = []
    for t in range(T):
        hs = gsb[:, t * D:(t + 1) * D]
        hd = gdb[:, t * D:(t + 1) * D]
        x = _relu(_dot(hs, w1a[0]) + _dot(hd, w1b[0]) + b1[0])
        x = _relu(_dot(x, w2[0]) + b2[0])
        m = _dot(x, w3[0]) + b3[0]
        ms.append(jnp.where(valid, m, 0.0))
    out[0, 0] = jnp.concatenate([ms[0], ms[1]], axis=-1)
    out[0, 1] = jnp.concatenate([ms[2], ms[3]], axis=-1)


def _run_edge_mlp(gs, gd, w1a, w1b, b1, w2, b2, w3, b3):
    def wspec(a):
        return pl.BlockSpec((1,) + a.shape[1:],
                            lambda s, e: (s,) + (0,) * (a.ndim - 1))
    in_specs = [
        pl.BlockSpec((1, EB, HROW), lambda s, e: (s, e, 0)),
        pl.BlockSpec((1, EB, HROW), lambda s, e: (s, e, 0)),
        wspec(w1a), wspec(w1b), wspec(b1), wspec(w2), wspec(b2),
        wspec(w3), wspec(b3),
    ]
    out_specs = pl.BlockSpec((1, 2, EB, 2 * D), lambda s, e: (s, 0, e, 0))
    return pl.pallas_call(
        _edge_body, grid=(SNUM, EP // EB), in_specs=in_specs,
        out_specs=out_specs,
        out_shape=jax.ShapeDtypeStruct((SNUM, 2, EP, 2 * D), F32),
    )(gs, gd, w1a, w1b, b1, w2, b2, w3, b3)


def _sc_scatter(msgs2d, dst_flat, zeros_row):
    mesh = plsc.VectorSubcoreMesh(core_axis_name="c", subcore_axis_name="s")

    @functools.partial(
        pl.kernel, mesh=mesh,
        out_type=jax.ShapeDtypeStruct((NC * SNUM * 2 * VP, 2 * D), F32),
        scratch_types=[
            pltpu.VMEM((CH,), I32),
            pltpu.VMEM((CH, 2 * D), F32),
            pltpu.VMEM((CH, 2 * D), F32),
            pltpu.VMEM_SHARED((VP, 2 * D), F32),
        ],
    )
    def scatter_k(m_hbm, dst_hbm, zero_hbm, out_hbm, idx_v, msg_v, zed_v,
                  acc_sh):
        cid = lax.axis_index("c")
        sid = lax.axis_index("s")
        wid = sid * NC + cid
        myrow = sid * VSLICE
        pltpu.sync_copy(zero_hbm, zed_v)
        for s in range(SNUM):
            for u in range(2):
                def zrow(z, carry):
                    pltpu.sync_copy(zed_v,
                                    acc_sh.at[pl.ds(myrow + z * CH, CH)])
                    return carry
                lax.fori_loop(0, VSLICE // CH, zrow, 0)
                plsc.subcore_barrier()

                def chunk(c, carry):
                    ci = wid * CPW + c
                    pltpu.sync_copy(dst_hbm.at[pl.ds(s * EP + ci * CH, CH)],
                                    idx_v)
                    pltpu.sync_copy(
                        m_hbm.at[pl.ds((s * 2 + u) * EP + ci * CH, CH)],
                        msg_v)
                    pltpu.sync_copy(msg_v, acc_sh.at[idx_v], add=True)
                    return carry
                lax.fori_loop(0, CPW, chunk, 0)
                plsc.subcore_barrier()
                pltpu.sync_copy(
                    acc_sh.at[pl.ds(myrow, VSLICE)],
                    out_hbm.at[pl.ds(((cid * SNUM + s) * 2 + u) * VP + myrow,
                                     VSLICE)])
                plsc.subcore_barrier()

    return scatter_k(msgs2d, dst_flat, zeros_row)


def _fin_body(hb, hcb, thb, vb, taub, pb, *refs):
    (wn1, bn1, wn2, bn2,
     wah, wam, watau, wath, ba, wa2, ba2,
     wd1, bd1, wd2, bd2, wd3, bd3, wd4, bd4, wd5, bd5) = [
        r[...] for r in refs[:21]]
    out = refs[21]
    taub = taub[...]
    agg2 = pb[0] + pb[1]

    msgs = []
    for t in range(T):
        row = []
        ht = hb[:, t, :]
        for s in range(SNUM):
            a = agg2[s, t // 2][:, (t % 2) * D:(t % 2 + 1) * D]
            x = jnp.concatenate([ht, a], axis=-1)
            x = _relu(_dot(x, wn1[s]) + bn1[s])
            row.append(_dot(x, wn2[s]) + bn2[s])
        msgs.append(row)

    vv = vb[...]
    radius = vv * taub
    smax = jnp.zeros_like(radius)
    for k in range(1, SNUM + 1):
        smax = smax + jax.nn.sigmoid(SHARP * (radius - float(k)))
    smax = jnp.maximum(smax - 1.0, 0.0)
    logmask_cols = []
    for t in range(T):
        st = smax[:, t:t + 1]
        for s in range(SNUM):
            mk = jax.nn.sigmoid(SHARP * (st - float(s)))
            logmask_cols.append(jnp.log(mk + 1e-9))
    logmask = jnp.concatenate(logmask_cols, axis=-1)

    hcur = hcb[...]
    base = _dot(hcur, wah) + _dot(thb[...], wath) + ba
    logit_cols = []
    for t in range(T):
        bt = base + taub[:, t:t + 1] * watau
        for s in range(SNUM):
            z = _relu(bt + _dot(msgs[t][s], wam))
            logit_cols.append(_dot(z, wa2) + ba2)
    logits = jnp.concatenate(logit_cols, axis=-1)

    masked = logits + logmask
    mx = jnp.max(masked, axis=-1, keepdims=True)
    ex = jnp.exp(masked - mx)
    w = ex / jnp.sum(ex, axis=-1, keepdims=True)

    aggf = jnp.zeros((hcur.shape[0], D), F32)
    i = 0
    for t in range(T):
        for s in range(SNUM):
            aggf = aggf + w[:, i:i + 1] * msgs[t][s]
            i += 1

    x = jnp.concatenate([hcur, aggf], axis=-1)
    x = jnp.tanh(_dot(x, wd1) + bd1)
    x = jnp.tanh(_dot(x, wd2) + bd2)
    x = jnp.tanh(_dot(x, wd3) + bd3)
    x = jnp.tanh(_dot(x, wd4) + bd4)
    out[...] = _dot(x, wd5) + bd5


def _run_finale(hh, hc, th, v, tau, partials, wts):
    nblk = VP // VB3
    full = lambda a: pl.BlockSpec(a.shape, lambda i: (0,) * a.ndim)
    in_specs = [
        pl.BlockSpec((VB3, T, D), lambda i: (i, 0, 0)),
        pl.BlockSpec((VB3, D), lambda i: (i, 0)),
        pl.BlockSpec((VB3, PROP), lambda i: (i, 0)),
        pl.BlockSpec((VB3, 1), lambda i: (i, 0)),
        full(tau),
        pl.BlockSpec((NC, SNUM, 2, VB3, 2 * D), lambda i: (0, 0, 0, i, 0)),
    ] + [full(w) for w in wts]
    return pl.pallas_call(
        _fin_body, grid=(nblk,), in_specs=in_specs,
        out_specs=pl.BlockSpec((VB3, 3), lambda i: (i, 0)),
        out_shape=jax.ShapeDtypeStruct((VP, 3), F32),
    )(hh, hc, th, v, tau, partials, *wts)


def _pad_rows(a, rows):
    return jnp.pad(a, ((0, rows - a.shape[0]), (0, 0)))


def kernel(dynamic_hist, reference_hist, dynamic_curr, reference_curr,
           constraint, stiffness, mass, tau_values, params,
           edge_index_s0, edge_index_s1, edge_index_s2):
    fh = jnp.concatenate(
        [dynamic_hist, reference_hist,
         jnp.zeros((T, V, 2), F32)], axis=-1)
    fh = jnp.pad(fh, ((0, 0), (0, VP - V), (0, 0)))
    fc = _pad_rows(jnp.concatenate(
        [dynamic_curr, reference_curr, jnp.zeros((V, 2), F32)], axis=-1), VP)
    fp = _pad_rows(jnp.concatenate(
        [constraint, stiffness, mass, jnp.zeros((V, 5), F32)], axis=-1), VP)

    p = params
    (wf1, bf1), (wf2, bf2) = p['feat_enc']
    (wp1, bp1), (wp2, bp2) = p['prop_enc']
    (wv1, bv1), (wv2, bv2) = p['vel']
    (wc1, bc1), (wc2, bc2) = p['cur_enc']
    wf1 = jnp.pad(wf1, ((0, 2), (0, 0)))
    wp1 = jnp.pad(wp1, ((0, 5), (0, 0)))
    wc1 = jnp.pad(wc1, ((0, 2), (0, 0)))
    r2 = lambda b: b.reshape(1, -1)
    enc_wts = [wf1, r2(bf1), wf2, r2(bf2), wp1, r2(bp1), wp2, r2(bp2),
               wv1, r2(bv1), wv2, r2(bv2), wc1, r2(bc1), wc2, r2(bc2)]

    hh, th, v, hc = _run_encoders(fh, fc, fp, enc_wts)
    h_table = hh.reshape(VP, HROW)

    eidx = [edge_index_s0, edge_index_s1, edge_index_s2]
    src = jnp.stack([jnp.pad(e[0].astype(I32), (0, EP - E)) for e in eidx])
    dst = jnp.stack([jnp.pad(e[1].astype(I32), (0, EP - E)) for e in eidx])

    gs, gd = _sc_gather(h_table, src.reshape(-1), dst.reshape(-1))

    w1a = jnp.stack([p['msea'][s]['edge'][0][0][:D] for s in range(SNUM)])
    w1b = jnp.stack([p['msea'][s]['edge'][0][0][D:] for s in range(SNUM)])
    b1 = jnp.stack([r2(p['msea'][s]['edge'][0][1]) for s in range(SNUM)])
    w2 = jnp.stack([p['msea'][s]['edge'][1][0] for s in range(SNUM)])
    b2 = jnp.stack([r2(p['msea'][s]['edge'][1][1]) for s in range(SNUM)])
    w3 = jnp.stack([p['msea'][s]['edge'][2][0] for s in range(SNUM)])
    b3 = jnp.stack([r2(p['msea'][s]['edge'][2][1]) for s in range(SNUM)])

    msgs = _run_edge_mlp(gs, gd, w1a, w1b, b1, w2, b2, w3, b3)

    partials = _sc_scatter(msgs.reshape(SNUM * 2 * EP, 2 * D),
                           dst.reshape(-1), jnp.zeros((CH, 2 * D), F32))
    partials = partials.reshape(NC, SNUM, 2, VP, 2 * D)

    wn1 = jnp.stack([p['msea'][s]['node'][0][0] for s in range(SNUM)])
    bn1 = jnp.stack([p['msea'][s]['node'][0][1] for s in range(SNUM)])[:, None, :]
    wn2 = jnp.stack([p['msea'][s]['node'][1][0] for s in range(SNUM)])
    bn2 = jnp.stack([p['msea'][s]['node'][1][1] for s in range(SNUM)])[:, None, :]
    wa, ba = p['attn'][0]
    wah = wa[:D]
    wam = wa[D:2 * D]
    watau = wa[2 * D:2 * D + 1]
    wath = wa[2 * D + 1:]
    wa2, ba2 = p['attn'][1]
    dyn = p['dyn']
    fin_wts = [wn1, bn1, wn2, bn2,
               wah, wam, watau, wath, r2(ba), wa2, r2(ba2),
               dyn[0][0], r2(dyn[0][1]), dyn[1][0], r2(dyn[1][1]),
               dyn[2][0], r2(dyn[2][1]), dyn[3][0], r2(dyn[3][1]),
               dyn[4][0], r2(dyn[4][1])]

    tau = tau_values.reshape(1, T).astype(F32)
    out = _run_finale(hh, hc, th, v, tau, partials, fin_wts)
    return out[:V]
